# baseline (speedup 1.0000x reference)
import functools
import jax
import jax.numpy as jnp
from jax import lax
from jax.experimental import pallas as pl
from jax.experimental.pallas import tpu as pltpu
from jax.experimental.pallas import tpu_sc as plsc

S = 16384
D = 128
SH = S // 2
NC, NS = 2, 16
NW = NC * NS
N_PAD = 102400
SB = 128
BLK = 512

_f32 = jnp.float32


def _k1_body(x_ref, m_ref, w2_ref, b2_ref, y2_ref, xm_ref):
    x = x_ref[...]
    y2_ref[...] = (
        jnp.dot(x, w2_ref[...].T, preferred_element_type=_f32) + b2_ref[...]
    )
    xm_ref[...] = x * m_ref[...]


def _tc_k1(x, m, W2, b2):
    n = x.shape[0]
    return pl.pallas_call(
        _k1_body,
        grid=(n // BLK,),
        in_specs=[
            pl.BlockSpec((BLK, D), lambda i: (i, 0)),
            pl.BlockSpec((BLK, 1), lambda i: (i, 0)),
            pl.BlockSpec((D, D), lambda i: (0, 0)),
            pl.BlockSpec((1, D), lambda i: (0, 0)),
        ],
        out_specs=[
            pl.BlockSpec((BLK, D), lambda i: (i, 0)),
            pl.BlockSpec((BLK, D), lambda i: (i, 0)),
        ],
        out_shape=[
            jax.ShapeDtypeStruct((n, D), _f32),
            jax.ShapeDtypeStruct((n, D), _f32),
        ],
    )(x, m, W2, b2)


def _k3_body(lr_ref, w1_ref, b1_ref, out_ref):
    out_ref[...] = (
        jnp.dot(lr_ref[...], w1_ref[...].T, preferred_element_type=_f32)
        + b1_ref[...]
    )


def _tc_k3(local_rep, W1, b1):
    return pl.pallas_call(
        _k3_body,
        grid=(S // BLK,),
        in_specs=[
            pl.BlockSpec((BLK, D), lambda i: (i, 0)),
            pl.BlockSpec((D, D), lambda i: (0, 0)),
            pl.BlockSpec((1, D), lambda i: (0, 0)),
        ],
        out_specs=pl.BlockSpec((BLK, D), lambda i: (i, 0)),
        out_shape=jax.ShapeDtypeStruct((S, D), _f32),
    )(local_rep, W1, b1)


def _k5_body(x_ref, y2_ref, g_ref, w3_ref, xw_ref):
    h = jax.nn.sigmoid(g_ref[...] + y2_ref[...])
    att = jnp.sum(h * w3_ref[...], axis=1, keepdims=True)
    xw_ref[...] = x_ref[...] * att


def _tc_k5(x, y2, lr1g, W3):
    n = x.shape[0]
    return pl.pallas_call(
        _k5_body,
        grid=(n // BLK,),
        in_specs=[
            pl.BlockSpec((BLK, D), lambda i: (i, 0)),
            pl.BlockSpec((BLK, D), lambda i: (i, 0)),
            pl.BlockSpec((BLK, D), lambda i: (i, 0)),
            pl.BlockSpec((1, D), lambda i: (0, 0)),
        ],
        out_specs=pl.BlockSpec((BLK, D), lambda i: (i, 0)),
        out_shape=jax.ShapeDtypeStruct((n, D), _f32),
    )(x, y2, lr1g, W3)


def _k7_body(lr_ref, g_ref, w4_ref, b4_ref, out_ref):
    rep = jnp.concatenate([lr_ref[...], g_ref[...]], axis=1)
    out_ref[...] = (
        jnp.dot(rep, w4_ref[...].T, preferred_element_type=_f32) + b4_ref[...]
    )


def _tc_k7(local_rep, glob_rep, W4, b4):
    return pl.pallas_call(
        _k7_body,
        grid=(S // BLK,),
        in_specs=[
            pl.BlockSpec((BLK, D), lambda i: (i, 0)),
            pl.BlockSpec((BLK, D), lambda i: (i, 0)),
            pl.BlockSpec((D, 2 * D), lambda i: (0, 0)),
            pl.BlockSpec((1, D), lambda i: (0, 0)),
        ],
        out_specs=pl.BlockSpec((BLK, D), lambda i: (i, 0)),
        out_shape=jax.ShapeDtypeStruct((S, D), _f32),
    )(local_rep, glob_rep, W4, b4)


@functools.cache
def _mesh():
    return plsc.VectorSubcoreMesh(
        core_axis_name="c", subcore_axis_name="s",
        num_cores=NC, num_subcores=NS,
    )


_ROWS_PER_SUB = N_PAD // NS
_SEG_CHUNK = SH // NS
_ROWS_PER_W = N_PAD // NW
_TRASH = SH
_SBLK = _ROWS_PER_SUB // SB
_GBLK = _ROWS_PER_W // SB
_NB = 3
_LOOK = 2
_GNB = 6
_GLOOK = 5


def _scatter_body(src_hbm, batch2_hbm, zeros_hbm, out_hbm, acc,
                  xbuf, ibuf, tbuf, s0, s1, s2):
    c = lax.axis_index("c")
    s = lax.axis_index("s")
    base = c * SH
    seg0 = s * _SEG_CHUNK
    dsem = [s0, s1, s2]

    pltpu.sync_copy(batch2_hbm.at[s], ibuf)
    pltpu.sync_copy(zeros_hbm, acc.at[pl.ds(seg0, _SEG_CHUNK), :])
    plsc.subcore_barrier()

    def hit(j):
        vf = ibuf[j, pl.ds(0, 16)]
        vl = ibuf[j, pl.ds(SB - 16, 16)]
        return jnp.logical_and(vl[15] >= base, vf[0] < base + SH)

    def desc(j):
        return pltpu.make_async_copy(
            src_hbm.at[s * _SBLK + j], xbuf.at[j % _NB], dsem[j % _NB]
        )

    for j in range(_LOOK):
        @pl.when(hit(j))
        def _(j=j):
            desc(j).start()

    for j in range(_SBLK):
        jn = j + _LOOK
        if jn < _SBLK:
            @pl.when(hit(jn))
            def _(jn=jn):
                desc(jn).start()

        @pl.when(hit(j))
        def _(j=j):
            for t in range(SB // 16):
                sl = pl.ds(t * 16, 16)
                v = ibuf[j, sl] - base
                ok = jnp.logical_and(v >= 0, v < SH)
                tbuf[sl] = jnp.where(ok, v, _TRASH)
            desc(j).wait()
            pltpu.sync_copy(xbuf.at[j % _NB], acc.at[tbuf], add=True)

    plsc.subcore_barrier()
    pltpu.sync_copy(
        acc.at[pl.ds(seg0, _SEG_CHUNK), :],
        out_hbm.at[c * NS + s],
    )


def _sc_scatter(src, batch2, zeros):
    return pl.kernel(
        _scatter_body,
        out_type=jax.ShapeDtypeStruct((NW, _SEG_CHUNK, D), _f32),
        mesh=_mesh(),
        scratch_types=[
            pltpu.VMEM_SHARED((SH + 8, D), _f32),
            pltpu.VMEM((_NB, SB, D), _f32),
            pltpu.VMEM((_SBLK, SB), jnp.int32),
            pltpu.VMEM((SB,), jnp.int32),
            pltpu.SemaphoreType.DMA,
            pltpu.SemaphoreType.DMA,
            pltpu.SemaphoreType.DMA,
        ],
    )(src, batch2, zeros)


def _gather_body(tab_hbm, batch2_hbm, out_hbm, rbuf, ibuf,
                 g0, g1, g2, g3, g4, g5, w0, w1, w2, w3, w4, w5):
    c = lax.axis_index("c")
    s = lax.axis_index("s")
    wid = c * NS + s
    gsem = [g0, g1, g2, g3, g4, g5]
    wsem = [w0, w1, w2, w3, w4, w5]

    pltpu.sync_copy(batch2_hbm.at[wid], ibuf)

    def issue(k):
        return pltpu.async_copy(
            tab_hbm.at[ibuf.at[k]], rbuf.at[k % _GNB], gsem[k % _GNB]
        )

    gdescs = {}
    wdescs = {}
    for k in range(_GLOOK):
        gdescs[k] = issue(k)

    for k in range(_GBLK):
        kn = k + _GLOOK
        if kn < _GBLK:
            if kn >= _GNB:
                wdescs.pop(kn - _GNB).wait()
            gdescs[kn] = issue(kn)
        gdescs.pop(k).wait()
        wdescs[k] = pltpu.async_copy(
            rbuf.at[k % _GNB], out_hbm.at[wid * _GBLK + k], wsem[k % _GNB]
        )

    for k in sorted(wdescs):
        wdescs.pop(k).wait()


def _sc_gather(tab, batch2):
    return pl.kernel(
        _gather_body,
        out_type=jax.ShapeDtypeStruct((N_PAD // SB, SB, D), _f32),
        mesh=_mesh(),
        scratch_types=[
            pltpu.VMEM((_GNB, SB, D), _f32),
            pltpu.VMEM((_GBLK, SB), jnp.int32),
            pltpu.SemaphoreType.DMA,
            pltpu.SemaphoreType.DMA,
            pltpu.SemaphoreType.DMA,
            pltpu.SemaphoreType.DMA,
            pltpu.SemaphoreType.DMA,
            pltpu.SemaphoreType.DMA,
            pltpu.SemaphoreType.DMA,
            pltpu.SemaphoreType.DMA,
            pltpu.SemaphoreType.DMA,
            pltpu.SemaphoreType.DMA,
            pltpu.SemaphoreType.DMA,
            pltpu.SemaphoreType.DMA,
        ],
    )(tab, batch2)


def kernel(x, batch, last_click_mask, W1, b1, W2, b2, W3, W4, b4):
    n = x.shape[0]
    pad = N_PAD - n
    x_p = jnp.pad(x, ((0, pad), (0, 0)))
    batch_p = jnp.pad(batch, (0, pad), constant_values=S - 1)
    batch2s = batch_p.reshape(NS, _SBLK, SB)
    batch2g = batch_p.reshape(NW, _GBLK, SB)
    m_p = jnp.pad(last_click_mask, (0, pad)).reshape(N_PAD, 1)
    zeros = jnp.zeros((_SEG_CHUNK, D), _f32)

    y2, xm = _tc_k1(x_p, m_p, W2, b2.reshape(1, D))
    local_rep = _sc_scatter(
        xm.reshape(N_PAD // SB, SB, D), batch2s, zeros).reshape(S, D)
    lr1 = _tc_k3(local_rep, W1, b1.reshape(1, D))
    lr1g = _sc_gather(lr1, batch2g).reshape(N_PAD, D)
    xw = _tc_k5(x_p, y2, lr1g, W3)
    glob_rep = _sc_scatter(
        xw.reshape(N_PAD // SB, SB, D), batch2s, zeros).reshape(S, D)
    return _tc_k7(local_rep, glob_rep, W4, b4.reshape(1, D))


# speedup vs baseline: 1.0047x; 1.0047x over previous
import functools
import jax
import jax.numpy as jnp
from jax import lax
from jax.experimental import pallas as pl
from jax.experimental.pallas import tpu as pltpu
from jax.experimental.pallas import tpu_sc as plsc

S = 16384
D = 128
SH = S // 2
NC, NS = 2, 16
NW = NC * NS
N_PAD = 102400
SB = 128
BLK = 512

_f32 = jnp.float32


def _k1_body(x_ref, m_ref, w2_ref, b2_ref, y2_ref, xm_ref):
    x = x_ref[...]
    y2_ref[...] = (
        jnp.dot(x, w2_ref[...].T, preferred_element_type=_f32) + b2_ref[...]
    )
    xm_ref[...] = x * m_ref[...]


def _tc_k1(x, m, W2, b2):
    n = x.shape[0]
    return pl.pallas_call(
        _k1_body,
        grid=(n // BLK,),
        in_specs=[
            pl.BlockSpec((BLK, D), lambda i: (i, 0)),
            pl.BlockSpec((BLK, 1), lambda i: (i, 0)),
            pl.BlockSpec((D, D), lambda i: (0, 0)),
            pl.BlockSpec((1, D), lambda i: (0, 0)),
        ],
        out_specs=[
            pl.BlockSpec((BLK, D), lambda i: (i, 0)),
            pl.BlockSpec((BLK, D), lambda i: (i, 0)),
        ],
        out_shape=[
            jax.ShapeDtypeStruct((n, D), _f32),
            jax.ShapeDtypeStruct((n, D), _f32),
        ],
    )(x, m, W2, b2)


def _k3_body(lr_ref, w1_ref, b1_ref, out_ref):
    out_ref[...] = (
        jnp.dot(lr_ref[...], w1_ref[...].T, preferred_element_type=_f32)
        + b1_ref[...]
    )


def _tc_k3(local_rep, W1, b1):
    return pl.pallas_call(
        _k3_body,
        grid=(S // BLK,),
        in_specs=[
            pl.BlockSpec((BLK, D), lambda i: (i, 0)),
            pl.BlockSpec((D, D), lambda i: (0, 0)),
            pl.BlockSpec((1, D), lambda i: (0, 0)),
        ],
        out_specs=pl.BlockSpec((BLK, D), lambda i: (i, 0)),
        out_shape=jax.ShapeDtypeStruct((S, D), _f32),
    )(local_rep, W1, b1)


def _k5_body(x_ref, y2_ref, g_ref, w3_ref, xw_ref):
    h = jax.nn.sigmoid(g_ref[...] + y2_ref[...])
    att = jnp.sum(h * w3_ref[...], axis=1, keepdims=True)
    xw_ref[...] = x_ref[...] * att


def _tc_k5(x, y2, lr1g, W3):
    n = x.shape[0]
    return pl.pallas_call(
        _k5_body,
        grid=(n // BLK,),
        in_specs=[
            pl.BlockSpec((BLK, D), lambda i: (i, 0)),
            pl.BlockSpec((BLK, D), lambda i: (i, 0)),
            pl.BlockSpec((BLK, D), lambda i: (i, 0)),
            pl.BlockSpec((1, D), lambda i: (0, 0)),
        ],
        out_specs=pl.BlockSpec((BLK, D), lambda i: (i, 0)),
        out_shape=jax.ShapeDtypeStruct((n, D), _f32),
    )(x, y2, lr1g, W3)


def _k7_body(lr_ref, g_ref, w4_ref, b4_ref, out_ref):
    rep = jnp.concatenate([lr_ref[...], g_ref[...]], axis=1)
    out_ref[...] = (
        jnp.dot(rep, w4_ref[...].T, preferred_element_type=_f32) + b4_ref[...]
    )


def _tc_k7(local_rep, glob_rep, W4, b4):
    return pl.pallas_call(
        _k7_body,
        grid=(S // BLK,),
        in_specs=[
            pl.BlockSpec((BLK, D), lambda i: (i, 0)),
            pl.BlockSpec((BLK, D), lambda i: (i, 0)),
            pl.BlockSpec((D, 2 * D), lambda i: (0, 0)),
            pl.BlockSpec((1, D), lambda i: (0, 0)),
        ],
        out_specs=pl.BlockSpec((BLK, D), lambda i: (i, 0)),
        out_shape=jax.ShapeDtypeStruct((S, D), _f32),
    )(local_rep, glob_rep, W4, b4)


@functools.cache
def _mesh():
    return plsc.VectorSubcoreMesh(
        core_axis_name="c", subcore_axis_name="s",
        num_cores=NC, num_subcores=NS,
    )


_ROWS_PER_SUB = N_PAD // NS
_SEG_CHUNK = SH // NS
_ROWS_PER_W = N_PAD // NW
_TRASH = SH
_SBLK = _ROWS_PER_SUB // SB
_GBLK = _ROWS_PER_W // SB
_NB = 3
_LOOK = 2
_GNB = 6
_GLOOK = 4


def _scatter_body(src_hbm, batch2_hbm, zeros_hbm, out_hbm, acc,
                  xbuf, ibuf, tbuf, s0, s1, s2):
    c = lax.axis_index("c")
    s = lax.axis_index("s")
    base = c * SH
    seg0 = s * _SEG_CHUNK
    dsem = [s0, s1, s2]

    pltpu.sync_copy(batch2_hbm.at[s], ibuf)
    pltpu.sync_copy(zeros_hbm, acc.at[pl.ds(seg0, _SEG_CHUNK), :])
    plsc.subcore_barrier()

    def hit(j):
        vf = ibuf[j, pl.ds(0, 16)]
        vl = ibuf[j, pl.ds(SB - 16, 16)]
        return jnp.logical_and(vl[15] >= base, vf[0] < base + SH)

    def desc(j):
        return pltpu.make_async_copy(
            src_hbm.at[s * _SBLK + j], xbuf.at[j % _NB], dsem[j % _NB]
        )

    for j in range(_LOOK):
        @pl.when(hit(j))
        def _(j=j):
            desc(j).start()

    for j in range(_SBLK):
        jn = j + _LOOK
        if jn < _SBLK:
            @pl.when(hit(jn))
            def _(jn=jn):
                desc(jn).start()

        @pl.when(hit(j))
        def _(j=j):
            for t in range(SB // 16):
                sl = pl.ds(t * 16, 16)
                v = ibuf[j, sl] - base
                ok = jnp.logical_and(v >= 0, v < SH)
                tbuf[sl] = jnp.where(ok, v, _TRASH)
            desc(j).wait()
            pltpu.sync_copy(xbuf.at[j % _NB], acc.at[tbuf], add=True)

    plsc.subcore_barrier()
    pltpu.sync_copy(
        acc.at[pl.ds(seg0, _SEG_CHUNK), :],
        out_hbm.at[c * NS + s],
    )


def _sc_scatter(src, batch2, zeros):
    return pl.kernel(
        _scatter_body,
        out_type=jax.ShapeDtypeStruct((NW, _SEG_CHUNK, D), _f32),
        mesh=_mesh(),
        scratch_types=[
            pltpu.VMEM_SHARED((SH + 8, D), _f32),
            pltpu.VMEM((_NB, SB, D), _f32),
            pltpu.VMEM((_SBLK, SB), jnp.int32),
            pltpu.VMEM((SB,), jnp.int32),
            pltpu.SemaphoreType.DMA,
            pltpu.SemaphoreType.DMA,
            pltpu.SemaphoreType.DMA,
        ],
    )(src, batch2, zeros)


def _gather_body(tab_hbm, batch2_hbm, out_hbm, rbuf, ibuf,
                 g0, g1, g2, g3, g4, g5, w0, w1, w2, w3, w4, w5):
    c = lax.axis_index("c")
    s = lax.axis_index("s")
    wid = c * NS + s
    gsem = [g0, g1, g2, g3, g4, g5]
    wsem = [w0, w1, w2, w3, w4, w5]

    pltpu.sync_copy(batch2_hbm.at[wid], ibuf)

    def issue(k):
        return pltpu.async_copy(
            tab_hbm.at[ibuf.at[k]], rbuf.at[k % _GNB], gsem[k % _GNB]
        )

    gdescs = {}
    wdescs = {}
    for k in range(_GLOOK):
        gdescs[k] = issue(k)

    for k in range(_GBLK):
        kn = k + _GLOOK
        if kn < _GBLK:
            if kn >= _GNB:
                wdescs.pop(kn - _GNB).wait()
            gdescs[kn] = issue(kn)
        gdescs.pop(k).wait()
        wdescs[k] = pltpu.async_copy(
            rbuf.at[k % _GNB], out_hbm.at[wid * _GBLK + k], wsem[k % _GNB]
        )

    for k in sorted(wdescs):
        wdescs.pop(k).wait()


def _sc_gather(tab, batch2):
    return pl.kernel(
        _gather_body,
        out_type=jax.ShapeDtypeStruct((N_PAD // SB, SB, D), _f32),
        mesh=_mesh(),
        scratch_types=[
            pltpu.VMEM((_GNB, SB, D), _f32),
            pltpu.VMEM((_GBLK, SB), jnp.int32),
            pltpu.SemaphoreType.DMA,
            pltpu.SemaphoreType.DMA,
            pltpu.SemaphoreType.DMA,
            pltpu.SemaphoreType.DMA,
            pltpu.SemaphoreType.DMA,
            pltpu.SemaphoreType.DMA,
            pltpu.SemaphoreType.DMA,
            pltpu.SemaphoreType.DMA,
            pltpu.SemaphoreType.DMA,
            pltpu.SemaphoreType.DMA,
            pltpu.SemaphoreType.DMA,
            pltpu.SemaphoreType.DMA,
        ],
    )(tab, batch2)


def kernel(x, batch, last_click_mask, W1, b1, W2, b2, W3, W4, b4):
    n = x.shape[0]
    pad = N_PAD - n
    x_p = jnp.pad(x, ((0, pad), (0, 0)))
    batch_p = jnp.pad(batch, (0, pad), constant_values=S - 1)
    batch2s = batch_p.reshape(NS, _SBLK, SB)
    batch2g = batch_p.reshape(NW, _GBLK, SB)
    m_p = jnp.pad(last_click_mask, (0, pad)).reshape(N_PAD, 1)
    zeros = jnp.zeros((_SEG_CHUNK, D), _f32)

    y2, xm = _tc_k1(x_p, m_p, W2, b2.reshape(1, D))
    local_rep = _sc_scatter(
        xm.reshape(N_PAD // SB, SB, D), batch2s, zeros).reshape(S, D)
    lr1 = _tc_k3(local_rep, W1, b1.reshape(1, D))
    lr1g = _sc_gather(lr1, batch2g).reshape(N_PAD, D)
    xw = _tc_k5(x_p, y2, lr1g, W3)
    glob_rep = _sc_scatter(
        xw.reshape(N_PAD // SB, SB, D), batch2s, zeros).reshape(S, D)
    return _tc_k7(local_rep, glob_rep, W4, b4.reshape(1, D))
